# Initial kernel scaffold; baseline (speedup 1.0000x reference)
import jax
import jax.numpy as jnp
from jax.experimental import pallas as pl


def kernel(item_table, item_idx):
    raise NotImplementedError("write your pallas kernel here")


# speedup vs baseline: 1.1101x; 1.1101x over previous
import jax
import jax.numpy as jnp
from jax import lax
from jax.experimental import pallas as pl
from jax.experimental.pallas import tpu as pltpu
from jax.experimental.pallas import tpu_sc as plsc

BATCH = 16384
HIST_LEN = 50
EMBEDDING_K = 32

_B = BATCH * HIST_LEN
_NW = 32
_PER_W = _B // _NW
_CH = 3200
_NCHUNK = _PER_W // _CH


def _gather_kernel(table_hbm, idx_hbm, out_hbm, idx_v, rows_v, sem):
    wid = lax.axis_index("s") * 2 + lax.axis_index("c")
    base = wid * _PER_W

    @pl.loop(0, _NCHUNK)
    def _(i):
        off = base + i * _CH
        pltpu.sync_copy(idx_hbm.at[pl.ds(off, _CH)], idx_v)
        pltpu.async_copy(table_hbm.at[idx_v], rows_v, sem).wait()
        pltpu.sync_copy(rows_v, out_hbm.at[pl.ds(off, _CH)])


@jax.jit
def _sc_gather(item_table, idx_flat):
    mesh = plsc.VectorSubcoreMesh(core_axis_name="c", subcore_axis_name="s")
    return pl.kernel(
        _gather_kernel,
        out_type=jax.ShapeDtypeStruct((_B, EMBEDDING_K), jnp.float32),
        mesh=mesh,
        compiler_params=pltpu.CompilerParams(use_tc_tiling_on_sc=False),
        scratch_types=[
            pltpu.VMEM((_CH,), jnp.int32),
            pltpu.VMEM((_CH, EMBEDDING_K), jnp.float32),
            pltpu.SemaphoreType.DMA,
        ],
    )(item_table, idx_flat)


def kernel(item_table, item_idx):
    idx_flat = item_idx.reshape(-1).astype(jnp.int32)
    out = _sc_gather(item_table, idx_flat)
    return out.reshape(BATCH, HIST_LEN, EMBEDDING_K)


# speedup vs baseline: 1.1140x; 1.0036x over previous
import jax
import jax.numpy as jnp
from jax import lax
from jax.experimental import pallas as pl
from jax.experimental.pallas import tpu as pltpu
from jax.experimental.pallas import tpu_sc as plsc

BATCH = 16384
HIST_LEN = 50
EMBEDDING_K = 32

_B = BATCH * HIST_LEN
_NW = 32
_PER_W = _B // _NW
_NBUF = 4
_LOOKAHEAD = 2
_CH = 800
_NCHUNK = _PER_W // _CH


def _gather_kernel(table_hbm, idx_hbm, out_hbm, *scratch):
    idx_bufs = scratch[0:_NBUF]
    row_bufs = scratch[_NBUF:2 * _NBUF]
    gsems = scratch[2 * _NBUF:3 * _NBUF]
    osems = scratch[3 * _NBUF:4 * _NBUF]

    wid = lax.axis_index("s") * 2 + lax.axis_index("c")
    base = wid * _PER_W

    def store_wait(chunk, b):
        pltpu.make_async_copy(
            row_bufs[b],
            out_hbm.at[pl.ds(base + chunk * _CH, _CH)],
            osems[b],
        ).wait()

    def retire(chunk, b):
        pltpu.make_async_copy(
            table_hbm.at[idx_bufs[b]], row_bufs[b], gsems[b]
        ).wait()
        pltpu.async_copy(
            row_bufs[b],
            out_hbm.at[pl.ds(base + chunk * _CH, _CH)],
            osems[b],
        )

    @pl.loop(0, _NCHUNK, step=_NBUF)
    def _(g):
        for b in range(_NBUF):
            i = g + b

            @pl.when(i >= _NBUF)
            def _():
                store_wait(i - _NBUF, b)

            pltpu.sync_copy(idx_hbm.at[pl.ds(base + i * _CH, _CH)],
                            idx_bufs[b])
            pltpu.async_copy(table_hbm.at[idx_bufs[b]], row_bufs[b],
                             gsems[b])

            @pl.when(i >= _LOOKAHEAD)
            def _():
                retire(i - _LOOKAHEAD, (b - _LOOKAHEAD) % _NBUF)

    for j in range(_NCHUNK - _LOOKAHEAD, _NCHUNK):
        retire(j, j % _NBUF)
    for j in range(_NCHUNK - _NBUF, _NCHUNK):
        store_wait(j, j % _NBUF)


@jax.jit
def _sc_gather(item_table, idx_flat):
    mesh = plsc.VectorSubcoreMesh(core_axis_name="c", subcore_axis_name="s")
    return pl.kernel(
        _gather_kernel,
        out_type=jax.ShapeDtypeStruct((_B, EMBEDDING_K), jnp.float32),
        mesh=mesh,
        compiler_params=pltpu.CompilerParams(use_tc_tiling_on_sc=False),
        scratch_types=(
            [pltpu.VMEM((_CH,), jnp.int32) for _ in range(_NBUF)]
            + [pltpu.VMEM((_CH, EMBEDDING_K), jnp.float32)
               for _ in range(_NBUF)]
            + [pltpu.SemaphoreType.DMA for _ in range(2 * _NBUF)]
        ),
    )(item_table, idx_flat)


def kernel(item_table, item_idx):
    idx_flat = item_idx.reshape(-1).astype(jnp.int32)
    out = _sc_gather(item_table, idx_flat)
    return out.reshape(BATCH, HIST_LEN, EMBEDDING_K)


# speedup vs baseline: 1.3231x; 1.1877x over previous
import jax
import jax.numpy as jnp
from jax import lax
from jax.experimental import pallas as pl
from jax.experimental.pallas import tpu as pltpu
from jax.experimental.pallas import tpu_sc as plsc

BATCH = 16384
HIST_LEN = 50
EMBEDDING_K = 32

_NW = 32
_CB = BATCH // _NW
_L = 16


def _gather_kernel(table_hbm, idx_hbm, out_hbm, idx_v, rows_v, trans_v, gsem):
    wid = lax.axis_index("s") * 2 + lax.axis_index("c")
    b0 = wid * _CB

    @pl.loop(0, HIST_LEN)
    def _(l):
        pltpu.sync_copy(idx_hbm.at[l, pl.ds(b0, _CB)], idx_v)
        pltpu.async_copy(table_hbm.at[idx_v], rows_v, gsem).wait()

        @pl.loop(0, EMBEDDING_K)
        def _(k):
            col = jnp.full((_L,), k, dtype=jnp.int32)

            @pl.loop(0, _CB, step=_L)
            def _(j0):
                rows = j0 + lax.iota(jnp.int32, _L)
                vals = plsc.load_gather(rows_v, [rows, col])
                trans_v[k, pl.ds(j0, _L)] = vals

        pltpu.sync_copy(trans_v, out_hbm.at[l, :, pl.ds(b0, _CB)])


@jax.jit
def _sc_gather(item_table, idx_t):
    mesh = plsc.VectorSubcoreMesh(core_axis_name="c", subcore_axis_name="s")
    return pl.kernel(
        _gather_kernel,
        out_type=jax.ShapeDtypeStruct((HIST_LEN, EMBEDDING_K, BATCH),
                                      jnp.float32),
        mesh=mesh,
        compiler_params=pltpu.CompilerParams(use_tc_tiling_on_sc=False,
                                             needs_layout_passes=False),
        scratch_types=[
            pltpu.VMEM((_CB,), jnp.int32),
            pltpu.VMEM((_CB, EMBEDDING_K), jnp.float32),
            pltpu.VMEM((EMBEDDING_K, _CB), jnp.float32),
            pltpu.SemaphoreType.DMA,
        ],
    )(item_table, idx_t)


def kernel(item_table, item_idx):
    idx_t = item_idx.T.astype(jnp.int32)
    out_t = _sc_gather(item_table, idx_t)
    return out_t.transpose(2, 0, 1)


# speedup vs baseline: 1.4652x; 1.1074x over previous
import jax
import jax.numpy as jnp
from jax import lax
from jax.experimental import pallas as pl
from jax.experimental.pallas import tpu as pltpu
from jax.experimental.pallas import tpu_sc as plsc

BATCH = 16384
HIST_LEN = 50
EMBEDDING_K = 32

_NW = 32
_CB = BATCH // _NW
_L = 16


def _transpose_block(rows_v, trans_v):
    iota = lax.iota(jnp.int32, _L)

    @pl.loop(0, _CB, step=_L)
    def _(j0):
        rows = j0 + iota
        for k in range(EMBEDDING_K):
            col = jnp.full((_L,), k, dtype=jnp.int32)
            vals = plsc.load_gather(rows_v, [rows, col])
            trans_v[k, pl.ds(j0, _L)] = vals


def _gather_kernel(table_hbm, idx_hbm, out_hbm,
                   idx_a, idx_b, rows_a, rows_b, trans_a, trans_b,
                   gsem_a, gsem_b, osem_a, osem_b):
    wid = lax.axis_index("s") * 2 + lax.axis_index("c")
    b0 = wid * _CB

    idx_bufs = (idx_a, idx_b)
    row_bufs = (rows_a, rows_b)
    trans_bufs = (trans_a, trans_b)
    gsems = (gsem_a, gsem_b)
    osems = (osem_a, osem_b)

    pltpu.sync_copy(idx_hbm.at[0, pl.ds(b0, _CB)], idx_a)
    pltpu.async_copy(table_hbm.at[idx_a], rows_a, gsem_a)

    @pl.loop(0, HIST_LEN, step=2)
    def _(l0):
        for p in range(2):
            l = l0 + p
            cur = p
            nxt = 1 - p

            @pl.when(l + 1 < HIST_LEN)
            def _():
                pltpu.sync_copy(idx_hbm.at[l + 1, pl.ds(b0, _CB)],
                                idx_bufs[nxt])
                pltpu.async_copy(table_hbm.at[idx_bufs[nxt]],
                                 row_bufs[nxt], gsems[nxt])

            pltpu.make_async_copy(table_hbm.at[idx_bufs[cur]],
                                  row_bufs[cur], gsems[cur]).wait()

            @pl.when(l >= 2)
            def _():
                pltpu.make_async_copy(
                    trans_bufs[cur],
                    out_hbm.at[l - 2, :, pl.ds(b0, _CB)],
                    osems[cur],
                ).wait()

            _transpose_block(row_bufs[cur], trans_bufs[cur])
            pltpu.async_copy(trans_bufs[cur],
                             out_hbm.at[l, :, pl.ds(b0, _CB)],
                             osems[cur])

    for l in (HIST_LEN - 2, HIST_LEN - 1):
        pltpu.make_async_copy(
            trans_bufs[l % 2],
            out_hbm.at[l, :, pl.ds(b0, _CB)],
            osems[l % 2],
        ).wait()


@jax.jit
def _sc_gather(item_table, idx_t):
    mesh = plsc.VectorSubcoreMesh(core_axis_name="c", subcore_axis_name="s")
    return pl.kernel(
        _gather_kernel,
        out_type=jax.ShapeDtypeStruct((HIST_LEN, EMBEDDING_K, BATCH),
                                      jnp.float32),
        mesh=mesh,
        compiler_params=pltpu.CompilerParams(use_tc_tiling_on_sc=False,
                                             needs_layout_passes=False),
        scratch_types=[
            pltpu.VMEM((_CB,), jnp.int32),
            pltpu.VMEM((_CB,), jnp.int32),
            pltpu.VMEM((_CB, EMBEDDING_K), jnp.float32),
            pltpu.VMEM((_CB, EMBEDDING_K), jnp.float32),
            pltpu.VMEM((EMBEDDING_K, _CB), jnp.float32),
            pltpu.VMEM((EMBEDDING_K, _CB), jnp.float32),
            pltpu.SemaphoreType.DMA,
            pltpu.SemaphoreType.DMA,
            pltpu.SemaphoreType.DMA,
            pltpu.SemaphoreType.DMA,
        ],
    )(item_table, idx_t)


def kernel(item_table, item_idx):
    idx_t = item_idx.T.astype(jnp.int32)
    out_t = _sc_gather(item_table, idx_t)
    return out_t.transpose(2, 0, 1)
